# Initial kernel scaffold; baseline (speedup 1.0000x reference)
import jax
import jax.numpy as jnp
from jax.experimental import pallas as pl


def kernel(x, edge_index, W_dense, b_dense, W_gat, att_src, att_dst, b_gat):
    raise NotImplementedError("write your pallas kernel here")


# speedup vs baseline: 13.0957x; 13.0957x over previous
import functools

import jax
import jax.numpy as jnp
from jax import lax
from jax.experimental import pallas as pl
from jax.experimental.pallas import tpu as pltpu
from jax.experimental.pallas import tpu_sc as plsc

N = 10000
E = 320000
IN_DIM = 128
EMBED_DIM = 128
OUT_DIM = 64

BM1 = 512
NPAD = 10240
K = 128
NW = 32
CHUNKS_PER_TILE = 81
EDGES_PAD = NW * CHUNKS_PER_TILE * K
ACC_W = 80
NACC = 10240
ROWS_PER_TILE = NACC // 16


def _encoder_body(x_ref, wdt_ref, b_ref, wgt_ref, avs_ref, avd_ref,
                  hw_ref, asrc_ref, adst_ref, ms_ref, md_ref):
    h = jnp.dot(x_ref[...], wdt_ref[...], preferred_element_type=jnp.float32)
    h = jnp.maximum(h + b_ref[...], 0.0)
    hw = jnp.dot(h, wgt_ref[...], preferred_element_type=jnp.float32)
    hw_ref[...] = hw
    a_s = jnp.dot(hw, avs_ref[...], preferred_element_type=jnp.float32)
    a_d = jnp.dot(hw, avd_ref[...], preferred_element_type=jnp.float32)
    asrc_ref[...] = a_s
    adst_ref[...] = a_d

    @pl.when(pl.program_id(0) == 0)
    def _():
        ms_ref[...] = jnp.full((1, 1), -1e30, jnp.float32)
        md_ref[...] = jnp.full((1, 1), -1e30, jnp.float32)

    ms_ref[...] = jnp.maximum(ms_ref[...], jnp.max(a_s))
    md_ref[...] = jnp.maximum(md_ref[...], jnp.max(a_d))


def _encoder(x_pad, wdt, b2d, wgt, att_s, att_d):
    grid = NPAD // BM1
    return pl.pallas_call(
        _encoder_body,
        grid=(grid,),
        in_specs=[
            pl.BlockSpec((BM1, IN_DIM), lambda i: (i, 0)),
            pl.BlockSpec((IN_DIM, EMBED_DIM), lambda i: (0, 0)),
            pl.BlockSpec((1, EMBED_DIM), lambda i: (0, 0)),
            pl.BlockSpec((EMBED_DIM, OUT_DIM), lambda i: (0, 0)),
            pl.BlockSpec((OUT_DIM, 1), lambda i: (0, 0)),
            pl.BlockSpec((OUT_DIM, 1), lambda i: (0, 0)),
        ],
        out_specs=[
            pl.BlockSpec((BM1, OUT_DIM), lambda i: (i, 0)),
            pl.BlockSpec((BM1, 1), lambda i: (i, 0)),
            pl.BlockSpec((BM1, 1), lambda i: (i, 0)),
            pl.BlockSpec((1, 1), lambda i: (0, 0)),
            pl.BlockSpec((1, 1), lambda i: (0, 0)),
        ],
        out_shape=[
            jax.ShapeDtypeStruct((NPAD, OUT_DIM), jnp.float32),
            jax.ShapeDtypeStruct((NPAD, 1), jnp.float32),
            jax.ShapeDtypeStruct((NPAD, 1), jnp.float32),
            jax.ShapeDtypeStruct((1, 1), jnp.float32),
            jax.ShapeDtypeStruct((1, 1), jnp.float32),
        ],
    )(x_pad, wdt, b2d, wgt, att_s, att_d)


def _sc_edge_body(src_hbm, dst_hbm, hw_hbm, asrc_hbm, adst_hbm, m_hbm,
                  out_hbm, srcv, dstv, rowbuf, av, adv, pv, sbuf, zbuf,
                  mvec, acc, sem1, sem2, sem3):
    cid = lax.axis_index("c")
    sid = lax.axis_index("s")
    wid = cid * 16 + sid

    z16 = jnp.zeros((16,), jnp.float32)

    def zrow(i, carry):
        for j in range(ACC_W // 16):
            zbuf[i, pl.ds(j * 16, 16)] = z16
        return carry

    lax.fori_loop(0, ROWS_PER_TILE // 2, zrow, 0)
    pltpu.sync_copy(zbuf, acc.at[pl.ds(sid * ROWS_PER_TILE, ROWS_PER_TILE // 2)])
    pltpu.sync_copy(
        zbuf,
        acc.at[pl.ds(sid * ROWS_PER_TILE + ROWS_PER_TILE // 2, ROWS_PER_TILE // 2)])
    pltpu.sync_copy(m_hbm, mvec)
    plsc.subcore_barrier()

    base0 = wid * (CHUNKS_PER_TILE * K)

    def chunk(g, carry):
        base = base0 + g * K
        pltpu.sync_copy(src_hbm.at[pl.ds(base, K)], srcv)
        pltpu.sync_copy(dst_hbm.at[pl.ds(base, K)], dstv)
        cp1 = pltpu.async_copy(hw_hbm.at[srcv], rowbuf, sem1)
        cp2 = pltpu.async_copy(asrc_hbm.at[srcv], av, sem2)
        cp3 = pltpu.async_copy(adst_hbm.at[dstv], adv, sem3)
        cp2.wait()
        cp3.wait()
        m = mvec[...]
        for j in range(K // 16):
            sl = pl.ds(j * 16, 16)
            e = av[sl] + adv[sl]
            e = jnp.where(e >= 0.0, e, 0.2 * e)
            pv[sl] = jnp.exp(e - m)
        cp1.wait()
        lid = lax.iota(jnp.int32, 16)

        def scale(gi, c2):
            pvec = pv[pl.ds(gi * 16, 16)]
            for e16 in range(16):
                ei = gi * 16 + e16
                ps = pvec[e16]
                for j in range(OUT_DIM // 16):
                    sl = pl.ds(j * 16, 16)
                    sbuf[ei, sl] = rowbuf[ei, sl] * ps
                sbuf[ei, pl.ds(OUT_DIM, 16)] = jnp.where(lid == 0, ps, 0.0)
            return c2

        lax.fori_loop(0, K // 16, scale, 0)
        pltpu.sync_copy(sbuf, acc.at[dstv], add=True)
        return carry

    lax.fori_loop(0, CHUNKS_PER_TILE, chunk, 0)
    plsc.subcore_barrier()

    pltpu.sync_copy(acc.at[pl.ds(sid * ROWS_PER_TILE, ROWS_PER_TILE)],
                    out_hbm.at[cid, pl.ds(sid * ROWS_PER_TILE, ROWS_PER_TILE)])


_sc_edge = functools.partial(
    pl.kernel,
    out_type=jax.ShapeDtypeStruct((2, NACC, ACC_W), jnp.float32),
    mesh=plsc.VectorSubcoreMesh(core_axis_name="c", subcore_axis_name="s"),
    compiler_params=pltpu.CompilerParams(use_tc_tiling_on_sc=False),
    scratch_types=[
        pltpu.VMEM((K,), jnp.int32),
        pltpu.VMEM((K,), jnp.int32),
        pltpu.VMEM((K, OUT_DIM), jnp.float32),
        pltpu.VMEM((K,), jnp.float32),
        pltpu.VMEM((K,), jnp.float32),
        pltpu.VMEM((K,), jnp.float32),
        pltpu.VMEM((K, ACC_W), jnp.float32),
        pltpu.VMEM((ROWS_PER_TILE // 2, ACC_W), jnp.float32),
        pltpu.VMEM((16,), jnp.float32),
        pltpu.VMEM_SHARED((NACC, ACC_W), jnp.float32),
        pltpu.SemaphoreType.DMA,
        pltpu.SemaphoreType.DMA,
        pltpu.SemaphoreType.DMA,
    ],
)(_sc_edge_body)


def _assemble_body(a0_ref, a1_ref, b_ref, out_ref):
    s = a0_ref[0] + a1_ref[0]
    numer = s[:, 0:OUT_DIM]
    denom = s[:, OUT_DIM:OUT_DIM + 1]
    out_ref[...] = numer / denom + b_ref[...]


def _assemble(acc2, b2d):
    bm = 400
    grid = N // bm
    return pl.pallas_call(
        _assemble_body,
        grid=(grid,),
        in_specs=[
            pl.BlockSpec((1, bm, ACC_W), lambda i: (0, i, 0)),
            pl.BlockSpec((1, bm, ACC_W), lambda i: (1, i, 0)),
            pl.BlockSpec((1, OUT_DIM), lambda i: (0, 0)),
        ],
        out_specs=pl.BlockSpec((bm, OUT_DIM), lambda i: (i, 0)),
        out_shape=jax.ShapeDtypeStruct((N, OUT_DIM), jnp.float32),
    )(acc2, acc2, b2d)


def _rec_body(l_ref, r_ref, out_ref):
    acc = lax.dot_general(l_ref[...], r_ref[...], (((1,), (1,)), ((), ())),
                          preferred_element_type=jnp.float32)
    out_ref[...] = jax.nn.sigmoid(acc)


def _rec(embed):
    bm = 512
    grid = pl.cdiv(N, bm)
    return pl.pallas_call(
        _rec_body,
        grid=(grid, grid),
        in_specs=[
            pl.BlockSpec((bm, OUT_DIM), lambda i, j: (i, 0)),
            pl.BlockSpec((bm, OUT_DIM), lambda i, j: (j, 0)),
        ],
        out_specs=pl.BlockSpec((bm, bm), lambda i, j: (i, j)),
        out_shape=jax.ShapeDtypeStruct((N, N), jnp.float32),
    )(embed, embed)


def kernel(x, edge_index, W_dense, b_dense, W_gat, att_src, att_dst, b_gat):
    x_pad = jnp.pad(x, ((0, NPAD - N), (0, 0)))
    hw, asrc, adst, ms, md = _encoder(
        x_pad, W_dense.T, b_dense[None, :], W_gat.T,
        att_src[:, None], att_dst[:, None])

    loop = jnp.arange(N, dtype=jnp.int32)
    padi = jnp.full((EDGES_PAD - E - N,), N, jnp.int32)
    srcp = jnp.concatenate([edge_index[0], loop, padi])
    dstp = jnp.concatenate([edge_index[1], loop, padi])

    s = ms[0, 0] + md[0, 0]
    mbound = jnp.where(s >= 0.0, s, 0.2 * s)
    marr = jnp.full((16,), mbound, jnp.float32)

    acc2 = _sc_edge(srcp, dstp, hw, asrc[:, 0], adst[:, 0], marr)
    embed = _assemble(acc2, b_gat[None, :])
    rec = _rec(embed)
    return rec, embed


# speedup vs baseline: 17.0085x; 1.2988x over previous
import functools

import jax
import jax.numpy as jnp
from jax import lax
from jax.experimental import pallas as pl
from jax.experimental.pallas import tpu as pltpu
from jax.experimental.pallas import tpu_sc as plsc

N = 10000
E = 320000
IN_DIM = 128
EMBED_DIM = 128
OUT_DIM = 64

BM1 = 512
NPAD = 10240
K = 128
NW = 32
CHUNKS_PER_TILE = 81
EDGES_PAD = NW * CHUNKS_PER_TILE * K
ACC_W = 80
NACC = 10240
ROWS_PER_TILE = NACC // 16
ZROWS = 80


def _encoder_body(x_ref, wdt_ref, b_ref, wgt_ref, avs_ref, avd_ref,
                  hw_ref, asrc_ref, adst_ref, ms_ref, md_ref):
    h = jnp.dot(x_ref[...], wdt_ref[...], preferred_element_type=jnp.float32)
    h = jnp.maximum(h + b_ref[...], 0.0)
    hw = jnp.dot(h, wgt_ref[...], preferred_element_type=jnp.float32)
    hw_ref[...] = hw
    a_s = jnp.dot(hw, avs_ref[...], preferred_element_type=jnp.float32)
    a_d = jnp.dot(hw, avd_ref[...], preferred_element_type=jnp.float32)
    asrc_ref[...] = a_s
    adst_ref[...] = a_d

    @pl.when(pl.program_id(0) == 0)
    def _():
        ms_ref[...] = jnp.full((1, 1), -1e30, jnp.float32)
        md_ref[...] = jnp.full((1, 1), -1e30, jnp.float32)

    ms_ref[...] = jnp.maximum(ms_ref[...], jnp.max(a_s))
    md_ref[...] = jnp.maximum(md_ref[...], jnp.max(a_d))


def _encoder(x_pad, wdt, b2d, wgt, att_s, att_d):
    grid = NPAD // BM1
    return pl.pallas_call(
        _encoder_body,
        grid=(grid,),
        in_specs=[
            pl.BlockSpec((BM1, IN_DIM), lambda i: (i, 0)),
            pl.BlockSpec((IN_DIM, EMBED_DIM), lambda i: (0, 0)),
            pl.BlockSpec((1, EMBED_DIM), lambda i: (0, 0)),
            pl.BlockSpec((EMBED_DIM, OUT_DIM), lambda i: (0, 0)),
            pl.BlockSpec((OUT_DIM, 1), lambda i: (0, 0)),
            pl.BlockSpec((OUT_DIM, 1), lambda i: (0, 0)),
        ],
        out_specs=[
            pl.BlockSpec((BM1, OUT_DIM), lambda i: (i, 0)),
            pl.BlockSpec((BM1, 1), lambda i: (i, 0)),
            pl.BlockSpec((BM1, 1), lambda i: (i, 0)),
            pl.BlockSpec((1, 1), lambda i: (0, 0)),
            pl.BlockSpec((1, 1), lambda i: (0, 0)),
        ],
        out_shape=[
            jax.ShapeDtypeStruct((NPAD, OUT_DIM), jnp.float32),
            jax.ShapeDtypeStruct((NPAD, 1), jnp.float32),
            jax.ShapeDtypeStruct((NPAD, 1), jnp.float32),
            jax.ShapeDtypeStruct((1, 1), jnp.float32),
            jax.ShapeDtypeStruct((1, 1), jnp.float32),
        ],
    )(x_pad, wdt, b2d, wgt, att_s, att_d)


def _sc_edge_body(sd_hbm, hw_hbm, asrc_hbm, adst_hbm, m_hbm,
                  out_hbm, idxall, rb0, rb1, av0, av1, ad0, ad1, pv,
                  sb0, sb1, zbuf, mvec, acc, gsem0, gsem1, ssem0, ssem1):
    cid = lax.axis_index("c")
    sid = lax.axis_index("s")
    wid = cid * 16 + sid

    idx_cp = pltpu.async_copy(sd_hbm.at[wid], idxall, gsem0)

    z16 = jnp.zeros((16,), jnp.float32)

    def zrow(i, carry):
        for j in range(ACC_W // 16):
            zbuf[i, pl.ds(j * 16, 16)] = z16
        return carry

    lax.fori_loop(0, ZROWS, zrow, 0)
    for r in range(ROWS_PER_TILE // ZROWS):
        pltpu.sync_copy(zbuf, acc.at[pl.ds(sid * ROWS_PER_TILE + r * ZROWS, ZROWS)])
    pltpu.sync_copy(m_hbm, mvec)
    idx_cp.wait()
    plsc.subcore_barrier()

    lid = lax.iota(jnp.int32, 16)

    def launch_gathers(g, rb, avb, adb, sem):
        pltpu.async_copy(hw_hbm.at[idxall.at[g, 0]], rb, sem)
        pltpu.async_copy(asrc_hbm.at[idxall.at[g, 0]], avb, sem)
        pltpu.async_copy(adst_hbm.at[idxall.at[g, 1]], adb, sem)

    def wait_gathers(rb, avb, adb, sem):
        pltpu.make_async_copy(hw_hbm.at[idxall.at[0, 0]], rb, sem).wait()
        pltpu.make_async_copy(asrc_hbm.at[idxall.at[0, 0]], avb, sem).wait()
        pltpu.make_async_copy(adst_hbm.at[idxall.at[0, 1]], adb, sem).wait()

    def wait_scatter(sb, sem):
        pltpu.make_async_copy(sb, acc.at[idxall.at[0, 1]], sem).wait()

    def process(g, rb, avb, adb, sb, ssem):
        m = mvec[...]
        for j in range(K // 16):
            sl = pl.ds(j * 16, 16)
            e = avb[sl] + adb[sl]
            e = jnp.where(e >= 0.0, e, 0.2 * e)
            pv[sl] = jnp.exp(e - m)

        def scale(gi, c2):
            pvec = pv[pl.ds(gi * 16, 16)]
            for e16 in range(16):
                ei = gi * 16 + e16
                ps = pvec[e16]
                for j in range(OUT_DIM // 16):
                    sl = pl.ds(j * 16, 16)
                    sb[ei, sl] = rb[ei, sl] * ps
                sb[ei, pl.ds(OUT_DIM, 16)] = jnp.where(lid == 0, ps, 0.0)
            return c2

        lax.fori_loop(0, K // 16, scale, 0)
        pltpu.async_copy(sb, acc.at[idxall.at[g, 1]], ssem, add=True)

    launch_gathers(0, rb0, av0, ad0, gsem0)

    def body(i, carry):
        a = 2 * i
        launch_gathers(a + 1, rb1, av1, ad1, gsem1)
        wait_gathers(rb0, av0, ad0, gsem0)

        @pl.when(i > 0)
        def _():
            wait_scatter(sb0, ssem0)

        process(a, rb0, av0, ad0, sb0, ssem0)
        launch_gathers(a + 2, rb0, av0, ad0, gsem0)
        wait_gathers(rb1, av1, ad1, gsem1)

        @pl.when(i > 0)
        def _():
            wait_scatter(sb1, ssem1)

        process(a + 1, rb1, av1, ad1, sb1, ssem1)
        return carry

    lax.fori_loop(0, (CHUNKS_PER_TILE - 1) // 2, body, 0)

    wait_gathers(rb0, av0, ad0, gsem0)
    wait_scatter(sb0, ssem0)
    process(CHUNKS_PER_TILE - 1, rb0, av0, ad0, sb0, ssem0)
    wait_scatter(sb1, ssem1)
    wait_scatter(sb0, ssem0)
    plsc.subcore_barrier()

    pltpu.sync_copy(acc.at[pl.ds(sid * ROWS_PER_TILE, ROWS_PER_TILE)],
                    out_hbm.at[cid, pl.ds(sid * ROWS_PER_TILE, ROWS_PER_TILE)])


_sc_edge = functools.partial(
    pl.kernel,
    out_type=jax.ShapeDtypeStruct((2, NACC, ACC_W), jnp.float32),
    mesh=plsc.VectorSubcoreMesh(core_axis_name="c", subcore_axis_name="s"),
    compiler_params=pltpu.CompilerParams(use_tc_tiling_on_sc=False),
    scratch_types=[
        pltpu.VMEM((CHUNKS_PER_TILE, 2, K), jnp.int32),
        pltpu.VMEM((K, OUT_DIM), jnp.float32),
        pltpu.VMEM((K, OUT_DIM), jnp.float32),
        pltpu.VMEM((K,), jnp.float32),
        pltpu.VMEM((K,), jnp.float32),
        pltpu.VMEM((K,), jnp.float32),
        pltpu.VMEM((K,), jnp.float32),
        pltpu.VMEM((K,), jnp.float32),
        pltpu.VMEM((K, ACC_W), jnp.float32),
        pltpu.VMEM((K, ACC_W), jnp.float32),
        pltpu.VMEM((ZROWS, ACC_W), jnp.float32),
        pltpu.VMEM((16,), jnp.float32),
        pltpu.VMEM_SHARED((NACC, ACC_W), jnp.float32),
        pltpu.SemaphoreType.DMA,
        pltpu.SemaphoreType.DMA,
        pltpu.SemaphoreType.DMA,
        pltpu.SemaphoreType.DMA,
    ],
)(_sc_edge_body)


def _assemble_body(a0_ref, a1_ref, b_ref, out_ref):
    s = a0_ref[0] + a1_ref[0]
    numer = s[:, 0:OUT_DIM]
    denom = s[:, OUT_DIM:OUT_DIM + 1]
    out_ref[...] = numer / denom + b_ref[...]


def _assemble(acc2, b2d):
    bm = 400
    grid = N // bm
    return pl.pallas_call(
        _assemble_body,
        grid=(grid,),
        in_specs=[
            pl.BlockSpec((1, bm, ACC_W), lambda i: (0, i, 0)),
            pl.BlockSpec((1, bm, ACC_W), lambda i: (1, i, 0)),
            pl.BlockSpec((1, OUT_DIM), lambda i: (0, 0)),
        ],
        out_specs=pl.BlockSpec((bm, OUT_DIM), lambda i: (i, 0)),
        out_shape=jax.ShapeDtypeStruct((N, OUT_DIM), jnp.float32),
    )(acc2, acc2, b2d)


def _rec_body(l_ref, r_ref, out_ref):
    acc = lax.dot_general(l_ref[...], r_ref[...], (((1,), (1,)), ((), ())),
                          preferred_element_type=jnp.float32)
    out_ref[...] = jax.nn.sigmoid(acc)


def _rec(embed):
    bm = 512
    grid = pl.cdiv(N, bm)
    return pl.pallas_call(
        _rec_body,
        grid=(grid, grid),
        in_specs=[
            pl.BlockSpec((bm, OUT_DIM), lambda i, j: (i, 0)),
            pl.BlockSpec((bm, OUT_DIM), lambda i, j: (j, 0)),
        ],
        out_specs=pl.BlockSpec((bm, bm), lambda i, j: (i, j)),
        out_shape=jax.ShapeDtypeStruct((N, N), jnp.float32),
    )(embed, embed)


def kernel(x, edge_index, W_dense, b_dense, W_gat, att_src, att_dst, b_gat):
    x_pad = jnp.pad(x, ((0, NPAD - N), (0, 0)))
    hw, asrc, adst, ms, md = _encoder(
        x_pad, W_dense.T, b_dense[None, :], W_gat.T,
        att_src[:, None], att_dst[:, None])

    loop = jnp.arange(N, dtype=jnp.int32)
    padi = jnp.full((EDGES_PAD - E - N,), N, jnp.int32)
    srcp = jnp.concatenate([edge_index[0], loop, padi])
    dstp = jnp.concatenate([edge_index[1], loop, padi])
    sd = jnp.concatenate(
        [srcp.reshape(NW, CHUNKS_PER_TILE, 1, K),
         dstp.reshape(NW, CHUNKS_PER_TILE, 1, K)], axis=2)

    s = ms[0, 0] + md[0, 0]
    mbound = jnp.where(s >= 0.0, s, 0.2 * s)
    marr = jnp.full((16,), mbound, jnp.float32)

    acc2 = _sc_edge(sd, hw, asrc[:, 0], adst[:, 0], marr)
    embed = _assemble(acc2, b_gat[None, :])
    rec = _rec(embed)
    return rec, embed
